# baseline (speedup 1.0000x reference)
import jax
import jax.numpy as jnp
from jax.experimental import pallas as pl
from jax.experimental.pallas import tpu as pltpu

B, N, F = 16, 2048, 128
NUM_CLASSES = 16
CH = 1024
NCH = N // CH


def _fused_gcn_kernel(x_ref, adj_ref, w1_ref, b1_ref, w2_ref, b2_ref,
                      w3_ref, b3_ref, w4_ref, b4_ref,
                      dec_ref, act_ref,
                      abf_ref, h1_ref, h2_ref, h3_ref):
    def layer(src, dst, w_ref, b_ref, act, cast_adj=False):
        t = jnp.dot(src, w_ref[...], preferred_element_type=jnp.float32)
        tb = t.astype(jnp.bfloat16)
        b = b_ref[...]
        for c in range(NCH):
            rows = pl.ds(c * CH, CH)
            if cast_adj:
                abf_ref[rows, :] = adj_ref[0, rows, :].astype(jnp.bfloat16)
            o = jnp.dot(abf_ref[rows, :], tb,
                        preferred_element_type=jnp.float32) + b
            dst[rows, :] = jnp.maximum(o, 0.0) if act else o

    layer(x_ref[0], h1_ref, w1_ref, b1_ref, True, cast_adj=True)
    layer(h1_ref[...], h2_ref, w2_ref, b2_ref, True)
    layer(h2_ref[...], h3_ref, w3_ref, b3_ref, True)

    t = jnp.dot(h3_ref[...], w4_ref[...], preferred_element_type=jnp.float32)
    tb = t.astype(jnp.bfloat16)
    b = b4_ref[...]
    lane = jax.lax.broadcasted_iota(jnp.int32, (CH, F), 1)
    is_cls = lane < NUM_CLASSES
    neg = jnp.float32(-1e30)
    for c in range(NCH):
        rows = pl.ds(c * CH, CH)
        dec = jnp.dot(abf_ref[rows, :], tb,
                      preferred_element_type=jnp.float32) + b
        dec_ref[0, rows, :] = dec
        m = jnp.max(jnp.where(is_cls, dec, neg), axis=-1, keepdims=True)
        e = jnp.exp(dec - m)
        denom = jnp.sum(jnp.where(is_cls, e, 0.0), axis=-1, keepdims=True)
        act_ref[0, rows, :] = jnp.where(is_cls, e / denom,
                                        jax.nn.sigmoid(dec))


def kernel(x, adj, W1, b1, W2, b2, W3, b3, W4, b4):
    b1r = b1.reshape(1, -1)
    b2r = b2.reshape(1, -1)
    b3r = b3.reshape(1, -1)
    b4r = b4.reshape(1, -1)

    full = lambda s: pl.BlockSpec(s, lambda i: (0,) * len(s))
    out_shape = [
        jax.ShapeDtypeStruct((B, N, F), jnp.float32),
        jax.ShapeDtypeStruct((B, N, F), jnp.float32),
    ]
    dec, act = pl.pallas_call(
        _fused_gcn_kernel,
        grid=(B,),
        in_specs=[
            pl.BlockSpec((1, N, F), lambda i: (i, 0, 0)),
            pl.BlockSpec((1, N, N), lambda i: (i, 0, 0)),
            full(W1.shape), full(b1r.shape),
            full(W2.shape), full(b2r.shape),
            full(W3.shape), full(b3r.shape),
            full(W4.shape), full(b4r.shape),
        ],
        out_specs=[
            pl.BlockSpec((1, N, F), lambda i: (i, 0, 0)),
            pl.BlockSpec((1, N, F), lambda i: (i, 0, 0)),
        ],
        out_shape=out_shape,
        scratch_shapes=[
            pltpu.VMEM((N, N), jnp.bfloat16),
            pltpu.VMEM((N, 30), jnp.float32),
            pltpu.VMEM((N, 18), jnp.float32),
            pltpu.VMEM((N, 30), jnp.float32),
        ],
    )(x, adj, W1, b1r, W2, b2r, W3, b3r, W4, b4r)
    return (dec, act)


# speedup vs baseline: 1.0102x; 1.0102x over previous
import jax
import jax.numpy as jnp
from jax.experimental import pallas as pl
from jax.experimental.pallas import tpu as pltpu

B, N, F = 16, 2048, 128
NUM_CLASSES = 16
CH = 512
NCH = N // CH


def _fused_gcn_kernel(x_ref, adj_ref, w1_ref, b1_ref, w2_ref, b2_ref,
                      w3_ref, b3_ref, w4_ref, b4_ref,
                      dec_ref, act_ref,
                      abf_ref, h1_ref, h2_ref, h3_ref):
    def layer(src, dst, w_ref, b_ref, act, cast_adj=False):
        t = jnp.dot(src, w_ref[...], preferred_element_type=jnp.float32)
        tb = t.astype(jnp.bfloat16)
        b = b_ref[...]
        for c in range(NCH):
            rows = pl.ds(c * CH, CH)
            if cast_adj:
                abf_ref[rows, :] = adj_ref[0, rows, :].astype(jnp.bfloat16)
            o = jnp.dot(abf_ref[rows, :], tb,
                        preferred_element_type=jnp.float32) + b
            dst[rows, :] = jnp.maximum(o, 0.0) if act else o

    layer(x_ref[0], h1_ref, w1_ref, b1_ref, True, cast_adj=True)
    layer(h1_ref[...], h2_ref, w2_ref, b2_ref, True)
    layer(h2_ref[...], h3_ref, w3_ref, b3_ref, True)

    t = jnp.dot(h3_ref[...], w4_ref[...], preferred_element_type=jnp.float32)
    tb = t.astype(jnp.bfloat16)
    b = b4_ref[...]
    lane = jax.lax.broadcasted_iota(jnp.int32, (CH, F), 1)
    is_cls = lane < NUM_CLASSES
    neg = jnp.float32(-1e30)
    for c in range(NCH):
        rows = pl.ds(c * CH, CH)
        dec = jnp.dot(abf_ref[rows, :], tb,
                      preferred_element_type=jnp.float32) + b
        dec_ref[0, rows, :] = dec
        m = jnp.max(jnp.where(is_cls, dec, neg), axis=-1, keepdims=True)
        e = jnp.exp(dec - m)
        denom = jnp.sum(jnp.where(is_cls, e, 0.0), axis=-1, keepdims=True)
        act_ref[0, rows, :] = jnp.where(is_cls, e / denom,
                                        jax.nn.sigmoid(dec))


def kernel(x, adj, W1, b1, W2, b2, W3, b3, W4, b4):
    b1r = b1.reshape(1, -1)
    b2r = b2.reshape(1, -1)
    b3r = b3.reshape(1, -1)
    b4r = b4.reshape(1, -1)

    full = lambda s: pl.BlockSpec(s, lambda i: (0,) * len(s))
    out_shape = [
        jax.ShapeDtypeStruct((B, N, F), jnp.float32),
        jax.ShapeDtypeStruct((B, N, F), jnp.float32),
    ]
    dec, act = pl.pallas_call(
        _fused_gcn_kernel,
        grid=(B,),
        in_specs=[
            pl.BlockSpec((1, N, F), lambda i: (i, 0, 0)),
            pl.BlockSpec((1, N, N), lambda i: (i, 0, 0)),
            full(W1.shape), full(b1r.shape),
            full(W2.shape), full(b2r.shape),
            full(W3.shape), full(b3r.shape),
            full(W4.shape), full(b4r.shape),
        ],
        out_specs=[
            pl.BlockSpec((1, N, F), lambda i: (i, 0, 0)),
            pl.BlockSpec((1, N, F), lambda i: (i, 0, 0)),
        ],
        out_shape=out_shape,
        scratch_shapes=[
            pltpu.VMEM((N, N), jnp.bfloat16),
            pltpu.VMEM((N, 30), jnp.float32),
            pltpu.VMEM((N, 18), jnp.float32),
            pltpu.VMEM((N, 30), jnp.float32),
        ],
    )(x, adj, W1, b1r, W2, b2r, W3, b3r, W4, b4r)
    return (dec, act)
